# Initial kernel scaffold; baseline (speedup 1.0000x reference)
import jax
import jax.numpy as jnp
from jax.experimental import pallas as pl


def kernel(x, word_table):
    raise NotImplementedError("write your pallas kernel here")


# speedup vs baseline: 1.7991x; 1.7991x over previous
import functools

import jax
import jax.numpy as jnp
from jax import lax
from jax.experimental import pallas as pl
from jax.experimental.pallas import tpu as pltpu
from jax.experimental.pallas import tpu_sc as plsc

VOCAB = 1000000
EMBED_DIM = 64
BATCH = 16384
HIST_LEN = 50

NUM_IDX = BATCH * HIST_LEN
IDX_ROW = 128
NUM_IDX_ROWS = NUM_IDX // IDX_ROW

NC = 2
NS = 16
NW = NC * NS
ROWS_PER_W = NUM_IDX_ROWS // NW

CHUNK_ROWS = 4
CHUNK = CHUNK_ROWS * IDX_ROW
NSTEPS = ROWS_PER_W // CHUNK_ROWS


_mesh = plsc.VectorSubcoreMesh(core_axis_name="c", subcore_axis_name="s")


@functools.partial(
    pl.kernel,
    mesh=_mesh,
    compiler_params=pltpu.CompilerParams(use_tc_tiling_on_sc=False),
    out_type=jax.ShapeDtypeStruct((NUM_IDX, EMBED_DIM), jnp.float32),
    scratch_types=[
        pltpu.VMEM((CHUNK_ROWS, IDX_ROW), jnp.int32),
        pltpu.VMEM((CHUNK, EMBED_DIM), jnp.float32),
        pltpu.SemaphoreType.DMA,
    ],
)
def _gather_kernel(idx_hbm, table_hbm, out_hbm, idx_v, rows_v, sem):
    wid = lax.axis_index("s") * NC + lax.axis_index("c")
    row_base = wid * ROWS_PER_W

    def step(i, carry):
        r0 = row_base + i * CHUNK_ROWS
        pltpu.sync_copy(idx_hbm.at[pl.ds(r0, CHUNK_ROWS)], idx_v)
        copies = [
            pltpu.async_copy(
                table_hbm.at[idx_v.at[j]],
                rows_v.at[pl.ds(j * IDX_ROW, IDX_ROW)],
                sem,
            )
            for j in range(CHUNK_ROWS)
        ]
        for c in copies:
            c.wait()
        pltpu.sync_copy(rows_v, out_hbm.at[pl.ds(r0 * IDX_ROW, CHUNK)])
        return carry

    lax.fori_loop(0, NSTEPS, step, 0)


def kernel(x, word_table):
    idx = x.reshape(NUM_IDX_ROWS, IDX_ROW).astype(jnp.int32)
    out = _gather_kernel(idx, word_table)
    return out.reshape(x.shape + (EMBED_DIM,))


# speedup vs baseline: 1.8740x; 1.0416x over previous
import functools

import jax
import jax.numpy as jnp
from jax import lax
from jax.experimental import pallas as pl
from jax.experimental.pallas import tpu as pltpu
from jax.experimental.pallas import tpu_sc as plsc

VOCAB = 1000000
EMBED_DIM = 64
BATCH = 16384
HIST_LEN = 50

NUM_IDX = BATCH * HIST_LEN
IDX_ROW = 128
NUM_IDX_ROWS = NUM_IDX // IDX_ROW

NC = 2
NS = 16
NW = NC * NS
ROWS_PER_W = NUM_IDX_ROWS // NW

NBUF = 4
CHUNK_ROWS = 2
CHUNK = CHUNK_ROWS * IDX_ROW
NSTEPS = ROWS_PER_W // CHUNK_ROWS
NGROUPS = NSTEPS // NBUF


_mesh = plsc.VectorSubcoreMesh(core_axis_name="c", subcore_axis_name="s")


@functools.partial(
    pl.kernel,
    mesh=_mesh,
    compiler_params=pltpu.CompilerParams(use_tc_tiling_on_sc=False),
    out_type=jax.ShapeDtypeStruct((NUM_IDX, EMBED_DIM), jnp.float32),
    scratch_types=[
        pltpu.VMEM((ROWS_PER_W, IDX_ROW), jnp.int32),
        pltpu.VMEM((NBUF, CHUNK, EMBED_DIM), jnp.float32),
        pltpu.SemaphoreType.DMA((NBUF,)),
        pltpu.SemaphoreType.DMA((NBUF,)),
    ],
)
def _gather_kernel(idx_hbm, table_hbm, out_hbm, idx_v, rows_v, gsem, wsem):
    wid = lax.axis_index("s") * NC + lax.axis_index("c")
    row_base = wid * ROWS_PER_W
    out_base = row_base * IDX_ROW

    pltpu.sync_copy(idx_hbm.at[pl.ds(row_base, ROWS_PER_W)], idx_v)

    def fire_gather(c, b):
        for j in range(CHUNK_ROWS):
            pltpu.async_copy(
                table_hbm.at[idx_v.at[c * CHUNK_ROWS + j]],
                rows_v.at[b, pl.ds(j * IDX_ROW, IDX_ROW)],
                gsem.at[b],
            )

    def wait_gather(c, b):
        for j in range(CHUNK_ROWS):
            pltpu.make_async_copy(
                table_hbm.at[idx_v.at[c * CHUNK_ROWS + j]],
                rows_v.at[b, pl.ds(j * IDX_ROW, IDX_ROW)],
                gsem.at[b],
            ).wait()

    def fire_write(c, b):
        pltpu.async_copy(
            rows_v.at[b], out_hbm.at[pl.ds(out_base + c * CHUNK, CHUNK)],
            wsem.at[b],
        )

    def wait_write(b):
        pltpu.make_async_copy(
            rows_v.at[b], out_hbm.at[pl.ds(out_base, CHUNK)], wsem.at[b]
        ).wait()

    for b in range(NBUF - 1):
        fire_gather(b, b)

    wait_gather(0, 0)
    fire_write(0, 0)
    fire_gather(NBUF - 1, NBUF - 1)

    for b in range(1, NBUF):
        wait_gather(b, b)
        fire_write(b, b)
        wait_write((b + NBUF - 1) % NBUF)
        fire_gather(b + NBUF - 1, (b + NBUF - 1) % NBUF)

    def group(g, carry):
        for b in range(NBUF):
            c = g * NBUF + b
            wait_gather(c, b)
            fire_write(c, b)
            wait_write((b + NBUF - 1) % NBUF)
            fire_gather(c + NBUF - 1, (b + NBUF - 1) % NBUF)
        return carry

    lax.fori_loop(1, NGROUPS - 1, group, 0)

    c0 = NSTEPS - NBUF
    wait_gather(c0, 0)
    fire_write(c0, 0)
    wait_write(NBUF - 1)
    fire_gather(NSTEPS - 1, NBUF - 1)
    for b in range(1, NBUF):
        wait_gather(c0 + b, b)
        fire_write(c0 + b, b)

    for b in range(NBUF):
        wait_write(b)


def kernel(x, word_table):
    idx = x.reshape(NUM_IDX_ROWS, IDX_ROW).astype(jnp.int32)
    out = _gather_kernel(idx, word_table)
    return out.reshape(x.shape + (EMBED_DIM,))
